# baseline (speedup 1.0000x reference)
import jax
import jax.numpy as jnp
from jax import lax
from jax.experimental import pallas as pl
from jax.experimental.pallas import tpu as pltpu
from jax.experimental.pallas import tpu_sc as plsc

RES = 128
N = RES ** 3
NC, NS = 2, 16
NW = NC * NS
C = N // NW
S = 4096
GROUPS = S // 16
BATCHES = S // 128
SUBCH = C // S
CNT_S = 8192

_CPARAMS = pltpu.CompilerParams(needs_layout_passes=False)


def _wid():
    return lax.axis_index("s") * NC + lax.axis_index("c")


def _count_body(mask_hbm, counts_hbm, mbuf, cbuf):
    wid = _wid()
    acc = jnp.zeros((16,), jnp.int32)
    for si in range(C // CNT_S):
        base = pl.multiple_of(wid * C + si * CNT_S, CNT_S)
        pltpu.sync_copy(mask_hbm.at[pl.ds(base, CNT_S)], mbuf)
        acc = lax.fori_loop(
            0, CNT_S // 16,
            lambda g, a: a + mbuf[pl.ds(g * 16, 16)],
            acc)
    cbuf[...] = jnp.full((16,), jnp.sum(acc), jnp.int32)
    pltpu.sync_copy(cbuf, counts_hbm.at[pl.ds(pl.multiple_of(wid * 16, 16), 16)])


def _main_body(mask_hbm, noise_hbm, params_hbm, counts_hbm,
               xw_hbm, occ_hbm,
               mbuf, nbuf, ysx, ysy, ysz, ostag, istag, pbuf, cntbuf, offbuf,
               sem_y, sem_o):
    wid = _wid()
    lanes = lax.iota(jnp.int32, 16)

    pltpu.sync_copy(params_hbm, pbuf)
    pltpu.sync_copy(counts_hbm, cntbuf)

    g0 = plsc.load_gather(cntbuf, [lanes * 16])
    g1 = plsc.load_gather(cntbuf, [256 + lanes * 16])
    tot0 = jnp.sum(g0)
    total = tot0 + jnp.sum(g1)
    e0 = plsc.cumsum(g0) - g0
    e1 = plsc.cumsum(g1) - g1 + tot0
    offbuf[pl.ds(0, 16)] = e0
    offbuf[pl.ds(16, 16)] = e1
    widv = jnp.full((16,), wid, jnp.int32)
    off1v = plsc.load_gather(offbuf, [widv])
    off0v = (total + wid * C) - off1v

    sxv, syv, szv = pbuf[0, :], pbuf[1, :], pbuf[2, :]
    oxv, oyv, ozv = pbuf[3, :], pbuf[4, :], pbuf[5, :]
    yfx, yfy, yfz = pbuf[6, :], pbuf[7, :], pbuf[8, :]

    chunk0 = wid * C

    def subchunk(si, carry):
        off1v, off0v = carry
        base = pl.multiple_of(chunk0 + si * S, S)
        pltpu.sync_copy(mask_hbm.at[pl.ds(base, S)], mbuf)
        pltpu.sync_copy(noise_hbm.at[pl.ds(base * 3, 3 * S)], nbuf)

        def group(g, c):
            o1, o0 = c
            mv = mbuf[pl.ds(g * 16, 16)]
            m = mv > 0
            incl = plsc.cumsum(mv)
            excl = incl - mv
            t1 = jnp.sum(mv)
            dst = jnp.where(m, o1 + excl, o0 + (lanes - excl))
            lrow = g * 16 + lanes
            vidx = base + lrow
            occv = jnp.where(m, vidx, 0)
            fi = (vidx >> 14).astype(jnp.float32)
            fj = ((vidx >> 7) & 127).astype(jnp.float32)
            fk = (vidx & 127).astype(jnp.float32)
            nx = plsc.load_gather(nbuf, [lrow * 3])
            ny = plsc.load_gather(nbuf, [lrow * 3 + 1])
            nz = plsc.load_gather(nbuf, [lrow * 3 + 2])
            yx = jnp.where(m, (fi + nx) * sxv + oxv, yfx)
            yy = jnp.where(m, (fj + ny) * syv + oyv, yfy)
            yz = jnp.where(m, (fk + nz) * szv + ozv, yfz)
            pos = g * 16
            ysx[pl.ds(pos, 16)] = yx
            ysy[pl.ds(pos, 16)] = yy
            ysz[pl.ds(pos, 16)] = yz
            ostag[pl.ds(pos, 16)] = occv
            row0 = jnp.full((16,), (g >> 3) * 4, jnp.int32)
            col = (g & 7) * 16 + lanes
            dst3 = dst * 3
            plsc.store_scatter(istag, [row0, col], dst)
            plsc.store_scatter(istag, [row0 + 1, col], dst3)
            plsc.store_scatter(istag, [row0 + 2, col], dst3 + 1)
            plsc.store_scatter(istag, [row0 + 3, col], dst3 + 2)
            return (o1 + t1, o0 + (16 - t1))

        off1v, off0v = lax.fori_loop(0, GROUPS, group, (off1v, off0v))

        copies = []
        for b in range(BATCHES):
            sl = pl.ds(b * 128, 128)
            copies.append(
                pltpu.async_copy(ostag.at[sl], occ_hbm.at[istag.at[4 * b]],
                                 sem_o))
            copies.append(
                pltpu.async_copy(ysx.at[sl], xw_hbm.at[istag.at[4 * b + 1]],
                                 sem_y))
            copies.append(
                pltpu.async_copy(ysy.at[sl], xw_hbm.at[istag.at[4 * b + 2]],
                                 sem_y))
            copies.append(
                pltpu.async_copy(ysz.at[sl], xw_hbm.at[istag.at[4 * b + 3]],
                                 sem_y))
        for cp in copies:
            cp.wait()
        return (off1v, off0v)

    lax.fori_loop(0, SUBCH, subchunk, (off1v, off0v))


def kernel(binary, noise, roi_aabb):
    mask = binary.reshape(-1).astype(jnp.int32)
    noise_flat = noise.reshape(-1)
    scale = (roi_aabb[3:] - roi_aabb[:3]) / jnp.float32(RES)
    offset = roi_aabb[:3]
    yfill = noise[0] * scale + offset
    params = jnp.tile(
        jnp.concatenate([scale, offset, yfill])[:, None], (1, 16))

    mesh = plsc.VectorSubcoreMesh(
        core_axis_name="c", subcore_axis_name="s",
        num_cores=NC, num_subcores=NS)

    count_k = pl.kernel(
        _count_body,
        out_type=jax.ShapeDtypeStruct((NW * 16,), jnp.int32),
        mesh=mesh,
        compiler_params=_CPARAMS,
        scratch_types=[
            pltpu.VMEM((CNT_S,), jnp.int32),
            pltpu.VMEM((16,), jnp.int32),
        ])
    counts = count_k(mask)

    main_k = pl.kernel(
        _main_body,
        out_type=(jax.ShapeDtypeStruct((3 * N,), jnp.float32),
                  jax.ShapeDtypeStruct((N,), jnp.int32)),
        mesh=mesh,
        compiler_params=_CPARAMS,
        scratch_types=[
            pltpu.VMEM((S,), jnp.int32),
            pltpu.VMEM((3 * S,), jnp.float32),
            pltpu.VMEM((S,), jnp.float32),
            pltpu.VMEM((S,), jnp.float32),
            pltpu.VMEM((S,), jnp.float32),
            pltpu.VMEM((S,), jnp.int32),
            pltpu.VMEM((4 * BATCHES, 128), jnp.int32),
            pltpu.VMEM((9, 16), jnp.float32),
            pltpu.VMEM((NW * 16,), jnp.int32),
            pltpu.VMEM((32,), jnp.int32),
            pltpu.SemaphoreType.DMA,
            pltpu.SemaphoreType.DMA,
        ])
    xw_flat, occ_indices = main_k(mask, noise_flat, params, counts)
    return xw_flat.reshape(N, 3), occ_indices
